# Initial kernel scaffold; baseline (speedup 1.0000x reference)
import jax
import jax.numpy as jnp
from jax.experimental import pallas as pl


def kernel(x, edge_index, weights, batch, params):
    raise NotImplementedError("write your pallas kernel here")


# speedup vs baseline: 5.6221x; 5.6221x over previous
import functools

import jax
import jax.numpy as jnp
from jax import lax
from jax.experimental import pallas as pl
from jax.experimental.pallas import tpu as pltpu
from jax.experimental.pallas import tpu_sc as plsc

N_NODES = 10000
E_EDGES = 320000
U_DIM = 128
EPS_BN = 1e-5

NC = 2
NS = 16
NW = NC * NS
EPT = E_EDGES // NW
BB = 128
NFULL = EPT // BB
TAIL = EPT - NFULL * BB
ROWS_PT = N_NODES // NS
_ROW_CHUNKS = ((0, 128), (128, 128), (256, 128), (384, 128), (512, 113))

_MESH = plsc.VectorSubcoreMesh(core_axis_name="c", subcore_axis_name="s")
_SC_PARAMS = pltpu.CompilerParams(needs_layout_passes=False,
                                  use_tc_tiling_on_sc=False)


def _deg_body(src_hbm, dst_hbm, w_hbm, out_hbm, src_v, dst_v, w_v, acc_v):
    c = lax.axis_index("c")
    s = lax.axis_index("s")
    wid = s * NC + c
    ebase = wid * EPT
    z16 = jnp.zeros((16,), jnp.float32)

    def zero_step(i, carry):
        acc_v[pl.ds(i * 16, 16)] = z16
        return carry

    lax.fori_loop(0, N_NODES // 16, zero_step, 0)
    pltpu.sync_copy(src_hbm.at[pl.ds(ebase, EPT)], src_v)
    pltpu.sync_copy(dst_hbm.at[pl.ds(ebase, EPT)], dst_v)
    pltpu.sync_copy(w_hbm.at[pl.ds(ebase, EPT)], w_v)

    def step(i, carry):
        sl = pl.ds(i * 16, 16)
        sv = src_v[sl]
        wv = jnp.where(sv == dst_v[sl], 0.0, w_v[sl])
        plsc.addupdate_scatter(acc_v, [sv], wv)
        return carry

    lax.fori_loop(0, EPT // 16, step, 0)
    pltpu.sync_copy(acc_v, out_hbm.at[wid])


_sc_deg = functools.partial(
    pl.kernel,
    out_type=jax.ShapeDtypeStruct((NW, N_NODES), jnp.float32),
    mesh=_MESH,
    compiler_params=_SC_PARAMS,
    scratch_types=[
        pltpu.VMEM((EPT,), jnp.int32),
        pltpu.VMEM((EPT,), jnp.int32),
        pltpu.VMEM((EPT,), jnp.float32),
        pltpu.VMEM((N_NODES,), jnp.float32),
    ],
)(_deg_body)


def _normw_body(src_hbm, dst_hbm, w_hbm, dinv_hbm, out_hbm,
                src_v, dst_v, w_v, dinv_v, nw_v):
    c = lax.axis_index("c")
    s = lax.axis_index("s")
    wid = s * NC + c
    ebase = wid * EPT
    pltpu.sync_copy(dinv_hbm, dinv_v)
    pltpu.sync_copy(src_hbm.at[pl.ds(ebase, EPT)], src_v)
    pltpu.sync_copy(dst_hbm.at[pl.ds(ebase, EPT)], dst_v)
    pltpu.sync_copy(w_hbm.at[pl.ds(ebase, EPT)], w_v)

    def step(i, carry):
        sl = pl.ds(i * 16, 16)
        sv = src_v[sl]
        dv = dst_v[sl]
        wv = jnp.where(sv == dv, 0.0, w_v[sl])
        da = plsc.load_gather(dinv_v, [sv])
        db = plsc.load_gather(dinv_v, [dv])
        nw_v[sl] = -(da * wv * db)
        return carry

    lax.fori_loop(0, EPT // 16, step, 0)
    pltpu.sync_copy(nw_v, out_hbm.at[pl.ds(ebase, EPT)])


_sc_normw = functools.partial(
    pl.kernel,
    out_type=jax.ShapeDtypeStruct((E_EDGES,), jnp.float32),
    mesh=_MESH,
    compiler_params=_SC_PARAMS,
    scratch_types=[
        pltpu.VMEM((EPT,), jnp.int32),
        pltpu.VMEM((EPT,), jnp.int32),
        pltpu.VMEM((EPT,), jnp.float32),
        pltpu.VMEM((N_NODES,), jnp.float32),
        pltpu.VMEM((EPT,), jnp.float32),
    ],
)(_normw_body)


def _lap_body(v_hbm, src_hbm, dst_hbm, nw_hbm, out_hbm,
              acc_sh, src_v, dst_v, tsrc_v, tdst_v, nw_v, rows_v, zbuf_v, sem):
    c = lax.axis_index("c")
    s = lax.axis_index("s")
    wid = s * NC + c
    ebase = wid * EPT
    rbase = s * ROWS_PT
    z16 = jnp.zeros((16,), jnp.float32)

    def zrow(r, carry):
        for j in range(U_DIM // 16):
            zbuf_v[r, pl.ds(j * 16, 16)] = z16
        return carry

    lax.fori_loop(0, BB, zrow, 0)
    for off, ln in _ROW_CHUNKS:
        pltpu.sync_copy(zbuf_v.at[pl.ds(0, ln)], acc_sh.at[pl.ds(rbase + off, ln)])
    plsc.subcore_barrier()

    def do_batch(bstart, blen, sidx, didx):
        pltpu.sync_copy(src_hbm.at[pl.ds(bstart, blen)], sidx)
        pltpu.sync_copy(dst_hbm.at[pl.ds(bstart, blen)], didx)
        pltpu.sync_copy(nw_hbm.at[pl.ds(bstart, blen)], nw_v.at[pl.ds(0, blen)])
        pltpu.async_copy(v_hbm.at[sidx], rows_v.at[pl.ds(0, blen)], sem).wait()

        def scale_group(g, carry):
            nw16 = nw_v[pl.ds(g * 16, 16)]
            for k in range(16):
                w = nw16[k]
                r = g * 16 + k
                for j in range(U_DIM // 16):
                    sl = pl.ds(j * 16, 16)
                    rows_v[r, sl] = rows_v[r, sl] * w
            return carry

        lax.fori_loop(0, blen // 16, scale_group, 0)
        pltpu.sync_copy(rows_v.at[pl.ds(0, blen)], acc_sh.at[didx], add=True)

    def batch_step(b, carry):
        do_batch(ebase + b * BB, BB, src_v, dst_v)
        return carry

    lax.fori_loop(0, NFULL, batch_step, 0)
    do_batch(ebase + NFULL * BB, TAIL, tsrc_v, tdst_v)

    plsc.subcore_barrier()
    pltpu.sync_copy(acc_sh.at[pl.ds(rbase, ROWS_PT)],
                    out_hbm.at[c, pl.ds(rbase, ROWS_PT)])


_sc_lap = functools.partial(
    pl.kernel,
    out_type=jax.ShapeDtypeStruct((NC, N_NODES, U_DIM), jnp.float32),
    mesh=_MESH,
    compiler_params=_SC_PARAMS,
    scratch_types=[
        pltpu.VMEM_SHARED((N_NODES, U_DIM), jnp.float32),
        pltpu.VMEM((BB,), jnp.int32),
        pltpu.VMEM((BB,), jnp.int32),
        pltpu.VMEM((TAIL,), jnp.int32),
        pltpu.VMEM((TAIL,), jnp.int32),
        pltpu.VMEM((BB,), jnp.float32),
        pltpu.VMEM((BB, U_DIM), jnp.float32),
        pltpu.VMEM((BB, U_DIM), jnp.float32),
        pltpu.SemaphoreType.DMA,
    ],
)(_lap_body)


def _dinv_body(parts_ref, out_ref):
    deg = jnp.sum(parts_ref[...], axis=0)
    out_ref[...] = jnp.where(deg > 0, lax.rsqrt(deg), 0.0)


_tc_dinv = pl.pallas_call(
    _dinv_body,
    out_shape=jax.ShapeDtypeStruct((N_NODES,), jnp.float32),
)


def _sum2_body(p_ref, o_ref):
    o_ref[...] = p_ref[0] + p_ref[1]


_tc_sum2 = pl.pallas_call(
    _sum2_body,
    grid=(5,),
    in_specs=[pl.BlockSpec((2, N_NODES // 5, U_DIM), lambda i: (0, i, 0))],
    out_specs=pl.BlockSpec((N_NODES // 5, U_DIM), lambda i: (i, 0)),
    out_shape=jax.ShapeDtypeStruct((N_NODES, U_DIM), jnp.float32),
)


def _combine_body(h_ref, tx1_ref, l2p_ref, w_ref, b_ref, g_ref, beta_ref, o_ref):
    h = h_ref[...]
    tx1 = tx1_ref[...]
    tx2 = 2.0 * (l2p_ref[0] + l2p_ref[1]) - h
    sacc = (jnp.dot(h, w_ref[0], preferred_element_type=jnp.float32)
            + jnp.dot(tx1, w_ref[1], preferred_element_type=jnp.float32)
            + jnp.dot(tx2, w_ref[2], preferred_element_type=jnp.float32)
            + b_ref[...])
    mu = jnp.mean(sacc, axis=0, keepdims=True)
    var = jnp.mean((sacc - mu) ** 2, axis=0, keepdims=True)
    y = (sacc - mu) * lax.rsqrt(var + EPS_BN) * g_ref[...] + beta_ref[...]
    o_ref[...] = jnp.maximum(y, 0.0)


_tc_combine = pl.pallas_call(
    _combine_body,
    out_shape=jax.ShapeDtypeStruct((N_NODES, U_DIM), jnp.float32),
)


def _final_body(h_ref, nw_ref, gw_ref, nb_ref, gb_ref, ln_ref, lg_ref):
    h = h_ref[...]
    ln_ref[...] = jnp.dot(h, nw_ref[...], preferred_element_type=jnp.float32) + nb_ref[0, 0]
    lg_ref[...] = (jnp.sum(h * gw_ref[...]) + gb_ref[0, 0])[None, None]


_tc_final = pl.pallas_call(
    _final_body,
    out_shape=(
        jax.ShapeDtypeStruct((N_NODES, 1), jnp.float32),
        jax.ShapeDtypeStruct((1, 1), jnp.float32),
    ),
)


def kernel(x, edge_index, weights, batch, params):
    del batch
    src = edge_index[0]
    dst = edge_index[1]

    deg_parts = _sc_deg(src, dst, weights)
    dinv = _tc_dinv(deg_parts)
    normw = _sc_normw(src, dst, weights, dinv)

    h = x
    for l in range(5):
        l1p = _sc_lap(h, src, dst, normw)
        tx1 = _tc_sum2(l1p)
        l2p = _sc_lap(tx1, src, dst, normw)
        h = _tc_combine(h, tx1, l2p, params[f"W{l}"],
                        params[f"b{l}"].reshape(1, U_DIM),
                        params[f"g{l}"].reshape(1, U_DIM),
                        params[f"beta{l}"].reshape(1, U_DIM))

    ln, lg = _tc_final(h,
                       params["node_w"].reshape(U_DIM, 1),
                       params["graph_w"].reshape(N_NODES, U_DIM),
                       params["node_b"].reshape(1, 1),
                       params["graph_b"].reshape(1, 1))
    logits_nodes = ln.reshape(1, N_NODES)
    logits_graph = lg.reshape(1,)
    return logits_nodes, logits_graph
